# baseline (speedup 1.0000x reference)
import functools

import jax
import jax.numpy as jnp
from jax import lax
from jax.experimental import pallas as pl
from jax.experimental.pallas import tpu as pltpu
from jax.experimental.pallas import tpu_sc as plsc

B = 16384
D = 64
HIDDEN = 128
EMB = 32
TILE_L = 128

NC, NS = 2, 16
NW = NC * NS
B_PER_W = B // NW
NBUF = 4


@functools.lru_cache(maxsize=None)
def _make_sc_gather():
    mesh = plsc.VectorSubcoreMesh(
        core_axis_name="c", subcore_axis_name="s", num_cores=NC, num_subcores=NS
    )

    pick_dnums = lax.GatherDimensionNumbers(
        offset_dims=(), collapsed_slice_dims=(0,), start_index_map=(0,)
    )

    @functools.partial(
        pl.kernel,
        out_type=jax.ShapeDtypeStruct((B * D,), jnp.float32),
        mesh=mesh,
        scratch_types=[
            pltpu.VMEM((B_PER_W,), jnp.int32),
            [pltpu.VMEM((D, TILE_L), jnp.float32)] * NBUF,
            pltpu.VMEM((B_PER_W * D,), jnp.float32),
            [pltpu.SemaphoreType.DMA] * NBUF,
        ],
        compiler_params=pltpu.CompilerParams(disable_bounds_checks=True),
    )
    def _sc_gather(idx_hbm, tableT_hbm, out_hbm, idx_v, bufs, rows_v, sems):
        wid = lax.axis_index("s") * NC + lax.axis_index("c")
        base = wid * B_PER_W
        pltpu.sync_copy(idx_hbm.at[pl.ds(base, B_PER_W)], idx_v)
        for i in range(B_PER_W // 16):
            idx_v[pl.ds(i * 16, 16)] = idx_v[pl.ds(i * 16, 16)] + 1

        lane_ids = jax.lax.iota(jnp.int32, 16)
        posmask = [lane_ids == m for m in range(16)]

        class fetch:
            def __init__(self, row, buf, sem):
                start = pl.multiple_of((row >> 7) * TILE_L, TILE_L)
                self.copies = [
                    pltpu.make_async_copy(
                        tableT_hbm.at[pl.ds(ti * 8, 8), pl.ds(start, TILE_L)],
                        buf.at[pl.ds(ti * 8, 8), :],
                        sem,
                    )
                    for ti in range(D // 8)
                ]

            def start(self):
                for cp in self.copies:
                    cp.start()

            def wait(self):
                for cp in self.copies:
                    cp.wait()

        def pick(vec16, pos):
            idx = jnp.full((16, 1), pos, dtype=jnp.int32)
            return lax.gather(
                vec16, idx, pick_dnums, (1,),
                mode=lax.GatherScatterMode.PROMISE_IN_BOUNDS,
            )

        def extract(buf, row, out_base):
            lane = row & (TILE_L - 1)
            win = (lane >> 4) * 16
            pos = lane & 15
            for k in range(D // 16):
                vs = []
                for m in range(16):
                    w = buf[k * 16 + m, pl.ds(win, 16)]
                    vs.append(
                        jnp.where(posmask[m], pick(w, pos), 0.0)
                    )
                while len(vs) > 1:
                    vs = [vs[i] + vs[i + 1] for i in range(0, len(vs), 2)]
                rows_v[pl.ds(out_base + k * 16, 16)] = vs[0]

        n_grp = 16 // NBUF

        def chunk(c, carry):
            co = c * 16
            rows16 = idx_v[pl.ds(co, 16)]
            for j in range(NBUF):
                fetch(rows16[j], bufs[j], sems[j]).start()
            for g in range(n_grp):
                for j in range(NBUF):
                    row = rows16[g * NBUF + j]
                    fetch(row, bufs[j], sems[j]).wait()
                    extract(bufs[j], row, (co + g * NBUF + j) * D)
                    if g + 1 < n_grp:
                        fetch(
                            rows16[(g + 1) * NBUF + j], bufs[j], sems[j]
                        ).start()
            return carry

        lax.fori_loop(0, B_PER_W // 16, chunk, 0)
        pltpu.sync_copy(rows_v, out_hbm.at[pl.ds(base * D, B_PER_W * D)])

    return _sc_gather


BB = 2048


def _mlp_body(x_ref, r_ref, w1a_ref, w1b_ref, b1_ref, w2_ref, b2_ref, o_ref):
    x = x_ref[...]
    r = r_ref[...]
    h = jnp.dot(x, w1a_ref[...], preferred_element_type=jnp.float32)
    h = h + (r - 3.0) * w1b_ref[...] + b1_ref[...]
    h = jnp.maximum(h, 0.0)
    o_ref[...] = (
        jnp.dot(h, w2_ref[...], preferred_element_type=jnp.float32)
        + b2_ref[...]
    )


def _mlp(gathered, rating_col, w1a, w1b, b1, w2, b2, interpret=False):
    grid = B // BB
    return pl.pallas_call(
        _mlp_body,
        grid=(grid,),
        in_specs=[
            pl.BlockSpec((BB, D), lambda i: (i, 0)),
            pl.BlockSpec((BB, 1), lambda i: (i, 0)),
            pl.BlockSpec((D, HIDDEN), lambda i: (0, 0)),
            pl.BlockSpec((1, HIDDEN), lambda i: (0, 0)),
            pl.BlockSpec((1, HIDDEN), lambda i: (0, 0)),
            pl.BlockSpec((HIDDEN, EMB), lambda i: (0, 0)),
            pl.BlockSpec((1, EMB), lambda i: (0, 0)),
        ],
        out_specs=pl.BlockSpec((BB, EMB), lambda i: (i, 0)),
        out_shape=jax.ShapeDtypeStruct((B, EMB), jnp.float32),
        interpret=interpret,
    )(gathered, rating_col, w1a, w1b, b1, w2, b2)


def kernel(book_id, avg_rating, table, W1, b1, W2, b2):
    gathered = _make_sc_gather()(book_id.astype(jnp.int32), table.T)
    gathered = gathered.reshape(B, D)
    return _mlp(
        gathered,
        avg_rating[:, None],
        W1[:D],
        W1[D:D + 1],
        b1[None, :],
        W2,
        b2[None, :],
    )


# speedup vs baseline: 1.0637x; 1.0637x over previous
import functools

import jax
import jax.numpy as jnp
from jax import lax
from jax.experimental import pallas as pl
from jax.experimental.pallas import tpu as pltpu
from jax.experimental.pallas import tpu_sc as plsc

B = 16384
D = 64
HIDDEN = 128
EMB = 32
TILE_L = 128

NC, NS = 2, 16
NW = NC * NS
B_PER_W = B // NW
NBUF = 4


@functools.lru_cache(maxsize=None)
def _make_sc_gather():
    mesh = plsc.VectorSubcoreMesh(
        core_axis_name="c", subcore_axis_name="s", num_cores=NC, num_subcores=NS
    )

    pick_dnums = lax.GatherDimensionNumbers(
        offset_dims=(), collapsed_slice_dims=(0,), start_index_map=(0,)
    )

    @functools.partial(
        pl.kernel,
        out_type=jax.ShapeDtypeStruct((B, D), jnp.float32),
        mesh=mesh,
        scratch_types=[
            pltpu.VMEM((B_PER_W,), jnp.int32),
            [pltpu.VMEM((D, TILE_L), jnp.float32)] * NBUF,
            pltpu.VMEM((B_PER_W, D), jnp.float32),
            [pltpu.SemaphoreType.DMA] * NBUF,
        ],
        compiler_params=pltpu.CompilerParams(disable_bounds_checks=True),
    )
    def _sc_gather(idx_hbm, tableT_hbm, out_hbm, idx_v, bufs, rows_v, sems):
        wid = lax.axis_index("s") * NC + lax.axis_index("c")
        base = wid * B_PER_W
        pltpu.sync_copy(idx_hbm.at[pl.ds(base, B_PER_W)], idx_v)
        for i in range(B_PER_W // 16):
            idx_v[pl.ds(i * 16, 16)] = idx_v[pl.ds(i * 16, 16)] + 1

        lane_ids = jax.lax.iota(jnp.int32, 16)
        posmask = [lane_ids == m for m in range(16)]

        def fetch(row, buf, sem):
            start = pl.multiple_of((row >> 7) * TILE_L, TILE_L)
            return pltpu.make_async_copy(
                tableT_hbm.at[:, pl.ds(start, TILE_L)], buf, sem
            )

        def pick(vec16, pos):
            idx = jnp.full((16, 1), pos, dtype=jnp.int32)
            return lax.gather(
                vec16, idx, pick_dnums, (1,),
                mode=lax.GatherScatterMode.PROMISE_IN_BOUNDS,
            )

        def extract(buf, row, out_base):
            lane = row & (TILE_L - 1)
            win = (lane >> 4) * 16
            pos = lane & 15
            for k in range(D // 16):
                vs = []
                for m in range(16):
                    w = buf[k * 16 + m, pl.ds(win, 16)]
                    vs.append(
                        jnp.where(posmask[m], pick(w, pos), 0.0)
                    )
                while len(vs) > 1:
                    vs = [vs[i] + vs[i + 1] for i in range(0, len(vs), 2)]
                rows_v[out_base, pl.ds(k * 16, 16)] = vs[0]

        n_grp = 16 // NBUF

        def chunk(c, carry):
            co = c * 16
            rows16 = idx_v[pl.ds(co, 16)]
            for j in range(NBUF):
                fetch(rows16[j], bufs[j], sems[j]).start()
            for g in range(n_grp):
                for j in range(NBUF):
                    row = rows16[g * NBUF + j]
                    fetch(row, bufs[j], sems[j]).wait()
                    extract(bufs[j], row, co + g * NBUF + j)
                    if g + 1 < n_grp:
                        fetch(
                            rows16[(g + 1) * NBUF + j], bufs[j], sems[j]
                        ).start()
            return carry

        lax.fori_loop(0, B_PER_W // 16, chunk, 0)
        pltpu.sync_copy(rows_v, out_hbm.at[pl.ds(base, B_PER_W)])

    return _sc_gather


BB = 2048


def _mlp_body(x_ref, r_ref, w1a_ref, w1b_ref, b1_ref, w2_ref, b2_ref, o_ref):
    x = x_ref[...]
    r = r_ref[...]
    h = jnp.dot(x, w1a_ref[...], preferred_element_type=jnp.float32)
    h = h + (r - 3.0) * w1b_ref[...] + b1_ref[...]
    h = jnp.maximum(h, 0.0)
    o_ref[...] = (
        jnp.dot(h, w2_ref[...], preferred_element_type=jnp.float32)
        + b2_ref[...]
    )


def _mlp(gathered, rating_col, w1a, w1b, b1, w2, b2, interpret=False):
    grid = B // BB
    return pl.pallas_call(
        _mlp_body,
        grid=(grid,),
        in_specs=[
            pl.BlockSpec((BB, D), lambda i: (i, 0)),
            pl.BlockSpec((BB, 1), lambda i: (i, 0)),
            pl.BlockSpec((D, HIDDEN), lambda i: (0, 0)),
            pl.BlockSpec((1, HIDDEN), lambda i: (0, 0)),
            pl.BlockSpec((1, HIDDEN), lambda i: (0, 0)),
            pl.BlockSpec((HIDDEN, EMB), lambda i: (0, 0)),
            pl.BlockSpec((1, EMB), lambda i: (0, 0)),
        ],
        out_specs=pl.BlockSpec((BB, EMB), lambda i: (i, 0)),
        out_shape=jax.ShapeDtypeStruct((B, EMB), jnp.float32),
        interpret=interpret,
    )(gathered, rating_col, w1a, w1b, b1, w2, b2)


def kernel(book_id, avg_rating, table, W1, b1, W2, b2):
    gathered = _make_sc_gather()(book_id.astype(jnp.int32), table.T)
    return _mlp(
        gathered,
        avg_rating[:, None],
        W1[:D],
        W1[D:D + 1],
        b1[None, :],
        W2,
        b2[None, :],
    )


# speedup vs baseline: 1.0646x; 1.0008x over previous
import functools

import jax
import jax.numpy as jnp
from jax import lax
from jax.experimental import pallas as pl
from jax.experimental.pallas import tpu as pltpu
from jax.experimental.pallas import tpu_sc as plsc

B = 16384
D = 64
HIDDEN = 128
EMB = 32
TILE_L = 128

NC, NS = 2, 16
NW = NC * NS
B_PER_W = B // NW
NBUF = 4


@functools.lru_cache(maxsize=None)
def _make_sc_gather():
    mesh = plsc.VectorSubcoreMesh(
        core_axis_name="c", subcore_axis_name="s", num_cores=NC, num_subcores=NS
    )

    pick_dnums = lax.GatherDimensionNumbers(
        offset_dims=(), collapsed_slice_dims=(0,), start_index_map=(0,)
    )

    @functools.partial(
        pl.kernel,
        out_type=jax.ShapeDtypeStruct((B, D), jnp.float32),
        mesh=mesh,
        scratch_types=[
            pltpu.VMEM((B_PER_W,), jnp.int32),
            [pltpu.VMEM((D, TILE_L), jnp.float32)] * NBUF,
            pltpu.VMEM((B_PER_W, D), jnp.float32),
            [pltpu.SemaphoreType.DMA] * NBUF,
        ],
        compiler_params=pltpu.CompilerParams(disable_bounds_checks=True),
    )
    def _sc_gather(idx_hbm, tableT_hbm, out_hbm, idx_v, bufs, rows_v, sems):
        wid = lax.axis_index("s") * NC + lax.axis_index("c")
        base = wid * B_PER_W
        pltpu.sync_copy(idx_hbm.at[pl.ds(base, B_PER_W)], idx_v)
        for i in range(B_PER_W // 16):
            idx_v[pl.ds(i * 16, 16)] = idx_v[pl.ds(i * 16, 16)] + 1

        lane_ids = jax.lax.iota(jnp.int32, 16)
        posmask = [lane_ids == m for m in range(16)]

        def _tile_copy(row, buf, sem):
            start = pl.multiple_of((row >> 7) * TILE_L, TILE_L)
            return pltpu.make_async_copy(
                tableT_hbm.at[:, pl.ds(start, TILE_L)], buf, sem
            )

        def fetch_start(row, buf, sem):
            _tile_copy(row, buf, sem).start()

        def fetch_wait(row, buf, sem):
            _tile_copy(row, buf, sem).wait()

        def pick(vec16, pos):
            idx = jnp.full((16, 1), pos, dtype=jnp.int32)
            return lax.gather(
                vec16, idx, pick_dnums, (1,),
                mode=lax.GatherScatterMode.PROMISE_IN_BOUNDS,
            )

        def extract(buf, row, out_base):
            lane = row & (TILE_L - 1)
            win = (lane >> 4) * 16
            pos = lane & 15
            for k in range(D // 16):
                vs = []
                for m in range(16):
                    w = buf[k * 16 + m, pl.ds(win, 16)]
                    vs.append(
                        jnp.where(posmask[m], pick(w, pos), 0.0)
                    )
                while len(vs) > 1:
                    vs = [vs[i] + vs[i + 1] for i in range(0, len(vs), 2)]
                rows_v[out_base, pl.ds(k * 16, 16)] = vs[0]

        n_grp = 16 // NBUF

        def chunk(c, carry):
            co = c * 16
            rows16 = idx_v[pl.ds(co, 16)]
            for j in range(NBUF):
                fetch_start(rows16[j], bufs[j], sems[j])
            for g in range(n_grp):
                for j in range(NBUF):
                    row = rows16[g * NBUF + j]
                    fetch_wait(row, bufs[j], sems[j])
                    extract(bufs[j], row, co + g * NBUF + j)
                    if g + 1 < n_grp:
                        fetch_start(
                            rows16[(g + 1) * NBUF + j], bufs[j], sems[j]
                        )
            return carry

        lax.fori_loop(0, B_PER_W // 16, chunk, 0)
        pltpu.sync_copy(rows_v, out_hbm.at[pl.ds(base, B_PER_W)])

    return _sc_gather


BB = 4096


def _mlp_body(x_ref, r_ref, w1a_ref, w1b_ref, b1_ref, w2_ref, b2_ref, o_ref):
    x = x_ref[...]
    r = r_ref[...]
    h = jnp.dot(x, w1a_ref[...], preferred_element_type=jnp.float32)
    h = h + (r - 3.0) * w1b_ref[...] + b1_ref[...]
    h = jnp.maximum(h, 0.0)
    o_ref[...] = (
        jnp.dot(h, w2_ref[...], preferred_element_type=jnp.float32)
        + b2_ref[...]
    )


def _mlp(gathered, rating_col, w1a, w1b, b1, w2, b2, interpret=False):
    grid = B // BB
    return pl.pallas_call(
        _mlp_body,
        grid=(grid,),
        in_specs=[
            pl.BlockSpec((BB, D), lambda i: (i, 0)),
            pl.BlockSpec((BB, 1), lambda i: (i, 0)),
            pl.BlockSpec((D, HIDDEN), lambda i: (0, 0)),
            pl.BlockSpec((1, HIDDEN), lambda i: (0, 0)),
            pl.BlockSpec((1, HIDDEN), lambda i: (0, 0)),
            pl.BlockSpec((HIDDEN, EMB), lambda i: (0, 0)),
            pl.BlockSpec((1, EMB), lambda i: (0, 0)),
        ],
        out_specs=pl.BlockSpec((BB, EMB), lambda i: (i, 0)),
        out_shape=jax.ShapeDtypeStruct((B, EMB), jnp.float32),
        interpret=interpret,
    )(gathered, rating_col, w1a, w1b, b1, w2, b2)


def kernel(book_id, avg_rating, table, W1, b1, W2, b2):
    gathered = _make_sc_gather()(book_id.astype(jnp.int32), table.T)
    return _mlp(
        gathered,
        avg_rating[:, None],
        W1[:D],
        W1[D:D + 1],
        b1[None, :],
        W2,
        b2[None, :],
    )
